# baseline (speedup 1.0000x reference)
import functools

import jax
import jax.numpy as jnp
from jax import lax
from jax.experimental import pallas as pl
from jax.experimental.pallas import tpu as pltpu
from jax.experimental.pallas import tpu_sc as plsc

D_H = 32
D_EH = 32
STEPS = 3
N_GRAPHS = 64

NC, NS = 2, 16
NW = NC * NS
LANES = 128
CHUNK = 1024


def _relu_mm_body(x_ref, w_ref, b_ref, o_ref):
    o_ref[...] = jax.nn.relu(
        jnp.dot(x_ref[...], w_ref[...], preferred_element_type=jnp.float32)
        + b_ref[...])


def _relu_mm(x, w, b, block_rows):
    n, fi = x.shape
    fo = w.shape[1]
    return pl.pallas_call(
        _relu_mm_body,
        grid=(n // block_rows,),
        in_specs=[pl.BlockSpec((block_rows, fi), lambda i: (i, 0)),
                  pl.BlockSpec((fi, fo), lambda i: (0, 0)),
                  pl.BlockSpec((1, fo), lambda i: (0, 0))],
        out_specs=pl.BlockSpec((block_rows, fo), lambda i: (i, 0)),
        out_shape=jax.ShapeDtypeStruct((n, fo), jnp.float32),
    )(x, w, b.reshape(1, fo))


def _msg_body(hs_ref, u_ref, w2p_ref, be2r_ref, o_ref):
    hs = hs_ref[...]
    t = jnp.dot(hs, w2p_ref[...], preferred_element_type=jnp.float32)
    acc = jnp.dot(hs, be2r_ref[...], preferred_element_type=jnp.float32)
    u = u_ref[...]
    for k in range(D_EH):
        acc = acc + u[:, k:k + 1] * t[:, k * D_H:(k + 1) * D_H]
    o_ref[...] = acc


def _msg(hs, u, w2p, be2r, block_rows):
    e_pad = hs.shape[0]
    return pl.pallas_call(
        _msg_body,
        grid=(e_pad // block_rows,),
        in_specs=[pl.BlockSpec((block_rows, D_H), lambda i: (i, 0)),
                  pl.BlockSpec((block_rows, D_EH), lambda i: (i, 0)),
                  pl.BlockSpec((D_H, D_EH * D_H), lambda i: (0, 0)),
                  pl.BlockSpec((D_H, D_H), lambda i: (0, 0))],
        out_specs=pl.BlockSpec((block_rows, D_H), lambda i: (i, 0)),
        out_shape=jax.ShapeDtypeStruct((e_pad, D_H), jnp.float32),
    )(hs, u, w2p, be2r)


def _gru_body(p_ref0, p_ref1, h_ref, wih_ref, bih_ref, whh_ref, bhh_ref,
              bconv_ref, o_ref):
    agg = p_ref0[0] + p_ref1[0] + bconv_ref[...]
    gi = jnp.dot(agg, wih_ref[...], preferred_element_type=jnp.float32) + bih_ref[...]
    h = h_ref[...]
    gh = jnp.dot(h, whh_ref[...], preferred_element_type=jnp.float32) + bhh_ref[...]
    r = jax.nn.sigmoid(gi[:, :D_H] + gh[:, :D_H])
    z = jax.nn.sigmoid(gi[:, D_H:2 * D_H] + gh[:, D_H:2 * D_H])
    nn = jnp.tanh(gi[:, 2 * D_H:] + r * gh[:, 2 * D_H:])
    o_ref[...] = (1.0 - z) * nn + z * h


def _gru(parts, h, w_ih, b_ih, w_hh, b_hh, b_conv, block_rows):
    n = h.shape[0]
    return pl.pallas_call(
        _gru_body,
        grid=(n // block_rows,),
        in_specs=[pl.BlockSpec((1, block_rows, D_H), lambda i: (0, i, 0)),
                  pl.BlockSpec((1, block_rows, D_H), lambda i: (1, i, 0)),
                  pl.BlockSpec((block_rows, D_H), lambda i: (i, 0)),
                  pl.BlockSpec((D_H, 3 * D_H), lambda i: (0, 0)),
                  pl.BlockSpec((1, 3 * D_H), lambda i: (0, 0)),
                  pl.BlockSpec((D_H, 3 * D_H), lambda i: (0, 0)),
                  pl.BlockSpec((1, 3 * D_H), lambda i: (0, 0)),
                  pl.BlockSpec((1, D_H), lambda i: (0, 0))],
        out_specs=pl.BlockSpec((block_rows, D_H), lambda i: (i, 0)),
        out_shape=jax.ShapeDtypeStruct((n, D_H), jnp.float32),
    )(parts, parts, h, w_ih, b_ih.reshape(1, -1), w_hh, b_hh.reshape(1, -1),
      b_conv.reshape(1, D_H))


def _readout_body(h_ref, n2g_ref, wf1_ref, bf1_ref, wf2_ref, bf2_ref,
                  wf3_ref, bf3_ref, o_ref, acc_ref):
    pi = pl.program_id(0)

    @pl.when(pi == 0)
    def _():
        acc_ref[...] = jnp.zeros_like(acc_ref)

    n2g = n2g_ref[0, 0, :]
    bn = n2g.shape[0]
    gids = lax.broadcasted_iota(jnp.int32, (N_GRAPHS, bn), 0)
    oh = jnp.where(gids == n2g[None, :], 1.0, 0.0)
    acc_ref[...] += jnp.dot(oh, h_ref[...], preferred_element_type=jnp.float32)

    @pl.when(pi == pl.num_programs(0) - 1)
    def _():
        g = acc_ref[...]
        p = jax.nn.relu(jnp.dot(g, wf1_ref[...],
                                preferred_element_type=jnp.float32) + bf1_ref[...])
        p = jax.nn.relu(jnp.dot(p, wf2_ref[...],
                                preferred_element_type=jnp.float32) + bf2_ref[...])
        o_ref[...] = jnp.dot(p, wf3_ref[...],
                             preferred_element_type=jnp.float32) + bf3_ref[...]


def _readout(h, n2g3d, wf1, bf1, wf2, bf2, wf3, bf3, block_rows):
    n = h.shape[0]
    nblk = n // block_rows
    return pl.pallas_call(
        _readout_body,
        grid=(nblk,),
        in_specs=[pl.BlockSpec((block_rows, D_H), lambda i: (i, 0)),
                  pl.BlockSpec((1, 1, block_rows), lambda i: (i, 0, 0)),
                  pl.BlockSpec((D_H, D_H), lambda i: (0, 0)),
                  pl.BlockSpec((1, D_H), lambda i: (0, 0)),
                  pl.BlockSpec((D_H, D_H), lambda i: (0, 0)),
                  pl.BlockSpec((1, D_H), lambda i: (0, 0)),
                  pl.BlockSpec((D_H, 1), lambda i: (0, 0)),
                  pl.BlockSpec((1, 1), lambda i: (0, 0))],
        out_specs=pl.BlockSpec((N_GRAPHS, 1), lambda i: (0, 0)),
        out_shape=jax.ShapeDtypeStruct((N_GRAPHS, 1), jnp.float32),
        scratch_shapes=[pltpu.VMEM((N_GRAPHS, D_H), jnp.float32)],
    )(h, n2g3d, wf1, bf1.reshape(1, D_H), wf2, bf2.reshape(1, D_H),
      wf3, bf3.reshape(1, 1))


def _sc_gather(table, idx2d, e_pad):
    rows_per_w = e_pad // NW
    n_chunks = rows_per_w // CHUNK
    mesh = plsc.VectorSubcoreMesh(core_axis_name="c", subcore_axis_name="s")

    @functools.partial(
        pl.kernel, mesh=mesh,
        compiler_params=pltpu.CompilerParams(use_tc_tiling_on_sc=False),
        out_type=jax.ShapeDtypeStruct((e_pad, D_H), jnp.float32),
        scratch_types=[pltpu.VMEM((8, LANES), jnp.int32),
                       pltpu.VMEM((CHUNK, D_H), jnp.float32),
                       pltpu.SemaphoreType.DMA],
    )
    def k(table_hbm, idx_hbm, out_hbm, idx_v, rows_v, sem):
        wid = lax.axis_index("c") * NS + lax.axis_index("s")
        for ci in range(n_chunks):
            base = pl.multiple_of(wid * rows_per_w + ci * CHUNK, 8)
            row = pl.multiple_of(wid * (rows_per_w // LANES) + ci * (CHUNK // LANES), 8)
            pltpu.sync_copy(idx_hbm.at[pl.ds(row, 8)], idx_v)
            copies = []
            for j in range(8):
                copies.append(pltpu.async_copy(
                    table_hbm.at[idx_v.at[j]],
                    rows_v.at[pl.ds(j * LANES, LANES)], sem))
            for cp in copies:
                cp.wait()
            pltpu.sync_copy(rows_v, out_hbm.at[pl.ds(base, CHUNK)])

    return k(table, idx2d)


def _sc_scatter_add(dst2d, msg, zeros, n, n_pad, e_pad):
    rows_per_w = e_pad // NW
    n_chunks = rows_per_w // CHUNK
    z_rows = n_pad // NS
    writers = 10
    o_rows = n // writers
    mesh = plsc.VectorSubcoreMesh(core_axis_name="c", subcore_axis_name="s")

    @functools.partial(
        pl.kernel, mesh=mesh,
        compiler_params=pltpu.CompilerParams(use_tc_tiling_on_sc=False),
        out_type=jax.ShapeDtypeStruct((NC, n, D_H), jnp.float32),
        scratch_types=[pltpu.VMEM((8, LANES), jnp.int32),
                       pltpu.VMEM((CHUNK, D_H), jnp.float32),
                       pltpu.VMEM_SHARED((n_pad, D_H), jnp.float32),
                       pltpu.SemaphoreType.DMA],
    )
    def k(dst_hbm, msg_hbm, zeros_hbm, out_hbm, idx_v, msg_v, agg_sh, sem):
        c = lax.axis_index("c")
        s = lax.axis_index("s")
        wid = c * NS + s
        zoff = pl.multiple_of(s * z_rows, 8)
        pltpu.sync_copy(zeros_hbm.at[pl.ds(zoff, z_rows)],
                        agg_sh.at[pl.ds(zoff, z_rows)])
        plsc.subcore_barrier()
        for ci in range(n_chunks):
            base = pl.multiple_of(wid * rows_per_w + ci * CHUNK, 8)
            row = pl.multiple_of(wid * (rows_per_w // LANES) + ci * (CHUNK // LANES), 8)
            pltpu.sync_copy(dst_hbm.at[pl.ds(row, 8)], idx_v)
            pltpu.sync_copy(msg_hbm.at[pl.ds(base, CHUNK)], msg_v)
            for j in range(8):
                pltpu.sync_copy(msg_v.at[pl.ds(j * LANES, LANES)],
                                agg_sh.at[idx_v.at[j]], add=True)
        plsc.subcore_barrier()

        @pl.when(s < writers)
        def _():
            ooff = pl.multiple_of(s * o_rows, 8)
            pltpu.sync_copy(agg_sh.at[pl.ds(ooff, o_rows)],
                            out_hbm.at[c].at[pl.ds(ooff, o_rows)])

    return k(dst2d, msg, zeros)


def kernel(x, edge_index, edge_attr, node2graph,
           W_enc, b_enc, We1, be1, We2, be2, b_conv,
           W_ih, b_ih, W_hh, b_hh,
           Wf1, bf1, Wf2, bf2, Wf3, bf3):
    n, _ = x.shape
    e = edge_attr.shape[0]
    e_pad = -(-e // (NW * CHUNK)) * (NW * CHUNK)
    n_pad = -(-(n + 1) // (NS * 8)) * (NS * 8)

    src = jnp.pad(edge_index[0], (0, e_pad - e)).reshape(e_pad // LANES, LANES)
    dst = jnp.pad(edge_index[1], (0, e_pad - e),
                  constant_values=n).reshape(e_pad // LANES, LANES)
    ea_pad = jnp.pad(edge_attr, ((0, e_pad - e), (0, 0)))

    w2p = We2.reshape(D_EH, D_H, D_H).transpose(1, 0, 2).reshape(D_H, D_EH * D_H)
    be2r = be2.reshape(D_H, D_H)
    zeros = jnp.zeros((n_pad, D_H), jnp.float32)

    out = _relu_mm(x, W_enc, b_enc, block_rows=2000)
    u = _relu_mm(ea_pad, We1, be1, block_rows=4096)
    h = out
    for _ in range(STEPS):
        hs = _sc_gather(out, src, e_pad)
        msg = _msg(hs, u, w2p, be2r, block_rows=1024)
        parts = _sc_scatter_add(dst, msg, zeros, n, n_pad, e_pad)
        h = _gru(parts, h, W_ih, b_ih, W_hh, b_hh, b_conv, block_rows=2000)
        out = h

    n2g3d = node2graph.reshape(5, 1, n // 5)
    return _readout(out, n2g3d, Wf1, bf1, Wf2, bf2, Wf3, bf3,
                    block_rows=n // 5)


# speedup vs baseline: 3.2525x; 3.2525x over previous
import functools

import jax
import jax.numpy as jnp
from jax import lax
from jax.experimental import pallas as pl
from jax.experimental.pallas import tpu as pltpu
from jax.experimental.pallas import tpu_sc as plsc

D_H = 32
D_EH = 32
STEPS = 3
N_GRAPHS = 64

NC, NS = 2, 16
NW = NC * NS
LANES = 128
CHUNK = 1024


def _relu_mm_body(x_ref, w_ref, b_ref, o_ref):
    o_ref[...] = jax.nn.relu(
        jnp.dot(x_ref[...], w_ref[...], preferred_element_type=jnp.float32)
        + b_ref[...])


def _relu_mm(x, w, b, block_rows):
    n, fi = x.shape
    fo = w.shape[1]
    return pl.pallas_call(
        _relu_mm_body,
        grid=(n // block_rows,),
        in_specs=[pl.BlockSpec((block_rows, fi), lambda i: (i, 0)),
                  pl.BlockSpec((fi, fo), lambda i: (0, 0)),
                  pl.BlockSpec((1, fo), lambda i: (0, 0))],
        out_specs=pl.BlockSpec((block_rows, fo), lambda i: (i, 0)),
        out_shape=jax.ShapeDtypeStruct((n, fo), jnp.float32),
    )(x, w, b.reshape(1, fo))


def _relu_mm_t_body(x_ref, w_ref, b_ref, o_ref):
    y = jax.nn.relu(
        jnp.dot(x_ref[...], w_ref[...], preferred_element_type=jnp.float32)
        + b_ref[...])
    o_ref[...] = jnp.concatenate(
        [y.T, jnp.ones((1, y.shape[0]), jnp.float32)], axis=0)


def _relu_mm_t(x, w, b, block_rows):
    n, fi = x.shape
    fo = w.shape[1]
    return pl.pallas_call(
        _relu_mm_t_body,
        grid=(n // block_rows,),
        in_specs=[pl.BlockSpec((block_rows, fi), lambda i: (i, 0)),
                  pl.BlockSpec((fi, fo), lambda i: (0, 0)),
                  pl.BlockSpec((1, fo), lambda i: (0, 0))],
        out_specs=pl.BlockSpec((fo + 1, block_rows), lambda i: (0, i)),
        out_shape=jax.ShapeDtypeStruct((fo + 1, n), jnp.float32),
    )(x, w, b.reshape(1, fo))


def _msg_body(hs_ref, ut_ref, wt_ref, o_ref):
    hst = hs_ref[...].T
    t3 = jnp.dot(wt_ref[...], hst, preferred_element_type=jnp.float32)
    te = hst.shape[1]
    ub = ut_ref[...].reshape(D_EH + 1, 1, te)
    p = t3.reshape(D_EH + 1, D_H, te) * ub
    msg3 = p.sum(axis=0)
    o_ref[...] = msg3.T


def _msg(hs, ut, wt, block_rows):
    e_pad = hs.shape[0]
    return pl.pallas_call(
        _msg_body,
        grid=(e_pad // block_rows,),
        in_specs=[pl.BlockSpec((block_rows, D_H), lambda i: (i, 0)),
                  pl.BlockSpec((D_EH + 1, block_rows), lambda i: (0, i)),
                  pl.BlockSpec(((D_EH + 1) * D_H, D_H), lambda i: (0, 0))],
        out_specs=pl.BlockSpec((block_rows, D_H), lambda i: (i, 0)),
        out_shape=jax.ShapeDtypeStruct((e_pad, D_H), jnp.float32),
    )(hs, ut, wt)


def _gru_body(p_ref0, p_ref1, h_ref, wih_ref, bih_ref, whh_ref, bhh_ref,
              bconv_ref, o_ref):
    agg = p_ref0[0] + p_ref1[0] + bconv_ref[...]
    gi = jnp.dot(agg, wih_ref[...], preferred_element_type=jnp.float32) + bih_ref[...]
    h = h_ref[...]
    gh = jnp.dot(h, whh_ref[...], preferred_element_type=jnp.float32) + bhh_ref[...]
    r = jax.nn.sigmoid(gi[:, :D_H] + gh[:, :D_H])
    z = jax.nn.sigmoid(gi[:, D_H:2 * D_H] + gh[:, D_H:2 * D_H])
    nn = jnp.tanh(gi[:, 2 * D_H:] + r * gh[:, 2 * D_H:])
    o_ref[...] = (1.0 - z) * nn + z * h


def _gru(parts, h, w_ih, b_ih, w_hh, b_hh, b_conv, block_rows):
    n = h.shape[0]
    return pl.pallas_call(
        _gru_body,
        grid=(n // block_rows,),
        in_specs=[pl.BlockSpec((1, block_rows, D_H), lambda i: (0, i, 0)),
                  pl.BlockSpec((1, block_rows, D_H), lambda i: (1, i, 0)),
                  pl.BlockSpec((block_rows, D_H), lambda i: (i, 0)),
                  pl.BlockSpec((D_H, 3 * D_H), lambda i: (0, 0)),
                  pl.BlockSpec((1, 3 * D_H), lambda i: (0, 0)),
                  pl.BlockSpec((D_H, 3 * D_H), lambda i: (0, 0)),
                  pl.BlockSpec((1, 3 * D_H), lambda i: (0, 0)),
                  pl.BlockSpec((1, D_H), lambda i: (0, 0))],
        out_specs=pl.BlockSpec((block_rows, D_H), lambda i: (i, 0)),
        out_shape=jax.ShapeDtypeStruct((n, D_H), jnp.float32),
    )(parts, parts, h, w_ih, b_ih.reshape(1, -1), w_hh, b_hh.reshape(1, -1),
      b_conv.reshape(1, D_H))


def _readout_body(h_ref, n2g_ref, wf1_ref, bf1_ref, wf2_ref, bf2_ref,
                  wf3_ref, bf3_ref, o_ref, acc_ref):
    pi = pl.program_id(0)

    @pl.when(pi == 0)
    def _():
        acc_ref[...] = jnp.zeros_like(acc_ref)

    n2g = n2g_ref[0, 0, :]
    bn = n2g.shape[0]
    gids = lax.broadcasted_iota(jnp.int32, (N_GRAPHS, bn), 0)
    oh = jnp.where(gids == n2g[None, :], 1.0, 0.0)
    acc_ref[...] += jnp.dot(oh, h_ref[...], preferred_element_type=jnp.float32)

    @pl.when(pi == pl.num_programs(0) - 1)
    def _():
        g = acc_ref[...]
        p = jax.nn.relu(jnp.dot(g, wf1_ref[...],
                                preferred_element_type=jnp.float32) + bf1_ref[...])
        p = jax.nn.relu(jnp.dot(p, wf2_ref[...],
                                preferred_element_type=jnp.float32) + bf2_ref[...])
        o_ref[...] = jnp.dot(p, wf3_ref[...],
                             preferred_element_type=jnp.float32) + bf3_ref[...]


def _readout(h, n2g3d, wf1, bf1, wf2, bf2, wf3, bf3, block_rows):
    n = h.shape[0]
    nblk = n // block_rows
    return pl.pallas_call(
        _readout_body,
        grid=(nblk,),
        in_specs=[pl.BlockSpec((block_rows, D_H), lambda i: (i, 0)),
                  pl.BlockSpec((1, 1, block_rows), lambda i: (i, 0, 0)),
                  pl.BlockSpec((D_H, D_H), lambda i: (0, 0)),
                  pl.BlockSpec((1, D_H), lambda i: (0, 0)),
                  pl.BlockSpec((D_H, D_H), lambda i: (0, 0)),
                  pl.BlockSpec((1, D_H), lambda i: (0, 0)),
                  pl.BlockSpec((D_H, 1), lambda i: (0, 0)),
                  pl.BlockSpec((1, 1), lambda i: (0, 0))],
        out_specs=pl.BlockSpec((N_GRAPHS, 1), lambda i: (0, 0)),
        out_shape=jax.ShapeDtypeStruct((N_GRAPHS, 1), jnp.float32),
        scratch_shapes=[pltpu.VMEM((N_GRAPHS, D_H), jnp.float32)],
    )(h, n2g3d, wf1, bf1.reshape(1, D_H), wf2, bf2.reshape(1, D_H),
      wf3, bf3.reshape(1, 1))


def _sc_gather(table, idx2d, e_pad):
    rows_per_w = e_pad // NW
    n_chunks = rows_per_w // CHUNK
    mesh = plsc.VectorSubcoreMesh(core_axis_name="c", subcore_axis_name="s")

    @functools.partial(
        pl.kernel, mesh=mesh,
        compiler_params=pltpu.CompilerParams(use_tc_tiling_on_sc=False),
        out_type=jax.ShapeDtypeStruct((e_pad, D_H), jnp.float32),
        scratch_types=[pltpu.VMEM((8, LANES), jnp.int32),
                       pltpu.VMEM((CHUNK, D_H), jnp.float32),
                       pltpu.SemaphoreType.DMA],
    )
    def k(table_hbm, idx_hbm, out_hbm, idx_v, rows_v, sem):
        wid = lax.axis_index("c") * NS + lax.axis_index("s")
        for ci in range(n_chunks):
            base = pl.multiple_of(wid * rows_per_w + ci * CHUNK, 8)
            row = pl.multiple_of(wid * (rows_per_w // LANES) + ci * (CHUNK // LANES), 8)
            pltpu.sync_copy(idx_hbm.at[pl.ds(row, 8)], idx_v)
            copies = []
            for j in range(8):
                copies.append(pltpu.async_copy(
                    table_hbm.at[idx_v.at[j]],
                    rows_v.at[pl.ds(j * LANES, LANES)], sem))
            for cp in copies:
                cp.wait()
            pltpu.sync_copy(rows_v, out_hbm.at[pl.ds(base, CHUNK)])

    return k(table, idx2d)


def _sc_scatter_add(dst2d, msg, zeros, n, n_pad, e_pad):
    rows_per_w = e_pad // NW
    n_chunks = rows_per_w // CHUNK
    z_rows = n_pad // NS
    writers = 10
    o_rows = n // writers
    mesh = plsc.VectorSubcoreMesh(core_axis_name="c", subcore_axis_name="s")

    @functools.partial(
        pl.kernel, mesh=mesh,
        compiler_params=pltpu.CompilerParams(use_tc_tiling_on_sc=False),
        out_type=jax.ShapeDtypeStruct((NC, n, D_H), jnp.float32),
        scratch_types=[pltpu.VMEM((8, LANES), jnp.int32),
                       pltpu.VMEM((CHUNK, D_H), jnp.float32),
                       pltpu.VMEM_SHARED((n_pad, D_H), jnp.float32),
                       pltpu.SemaphoreType.DMA],
    )
    def k(dst_hbm, msg_hbm, zeros_hbm, out_hbm, idx_v, msg_v, agg_sh, sem):
        c = lax.axis_index("c")
        s = lax.axis_index("s")
        wid = c * NS + s
        zoff = pl.multiple_of(s * z_rows, 8)
        pltpu.sync_copy(zeros_hbm.at[pl.ds(zoff, z_rows)],
                        agg_sh.at[pl.ds(zoff, z_rows)])
        plsc.subcore_barrier()
        for ci in range(n_chunks):
            base = pl.multiple_of(wid * rows_per_w + ci * CHUNK, 8)
            row = pl.multiple_of(wid * (rows_per_w // LANES) + ci * (CHUNK // LANES), 8)
            pltpu.sync_copy(dst_hbm.at[pl.ds(row, 8)], idx_v)
            pltpu.sync_copy(msg_hbm.at[pl.ds(base, CHUNK)], msg_v)
            for j in range(8):
                pltpu.sync_copy(msg_v.at[pl.ds(j * LANES, LANES)],
                                agg_sh.at[idx_v.at[j]], add=True)
        plsc.subcore_barrier()

        @pl.when(s < writers)
        def _():
            ooff = pl.multiple_of(s * o_rows, 8)
            pltpu.sync_copy(agg_sh.at[pl.ds(ooff, o_rows)],
                            out_hbm.at[c].at[pl.ds(ooff, o_rows)])

    return k(dst2d, msg, zeros)


def kernel(x, edge_index, edge_attr, node2graph,
           W_enc, b_enc, We1, be1, We2, be2, b_conv,
           W_ih, b_ih, W_hh, b_hh,
           Wf1, bf1, Wf2, bf2, Wf3, bf3):
    n, _ = x.shape
    e = edge_attr.shape[0]
    e_pad = -(-e // (NW * CHUNK)) * (NW * CHUNK)
    n_pad = -(-(n + 1) // (NS * 8)) * (NS * 8)

    src = jnp.pad(edge_index[0], (0, e_pad - e)).reshape(e_pad // LANES, LANES)
    dst = jnp.pad(edge_index[1], (0, e_pad - e),
                  constant_values=n).reshape(e_pad // LANES, LANES)
    ea_pad = jnp.pad(edge_attr, ((0, e_pad - e), (0, 0)))

    wt = jnp.concatenate(
        [We2.reshape(D_EH, D_H, D_H).transpose(0, 2, 1).reshape(D_EH * D_H, D_H),
         be2.reshape(D_H, D_H).T], axis=0)
    zeros = jnp.zeros((n_pad, D_H), jnp.float32)

    out = _relu_mm(x, W_enc, b_enc, block_rows=2000)
    ut = _relu_mm_t(ea_pad, We1, be1, block_rows=4096)
    h = out
    for _ in range(STEPS):
        hs = _sc_gather(out, src, e_pad)
        msg = _msg(hs, ut, wt, block_rows=1024)
        parts = _sc_scatter_add(dst, msg, zeros, n, n_pad, e_pad)
        h = _gru(parts, h, W_ih, b_ih, W_hh, b_hh, b_conv, block_rows=2000)
        out = h

    n2g3d = node2graph.reshape(5, 1, n // 5)
    return _readout(out, n2g3d, Wf1, bf1, Wf2, bf2, Wf3, bf3,
                    block_rows=n // 5)


# speedup vs baseline: 3.6933x; 1.1355x over previous
import functools

import jax
import jax.numpy as jnp
from jax import lax
from jax.experimental import pallas as pl
from jax.experimental.pallas import tpu as pltpu
from jax.experimental.pallas import tpu_sc as plsc

D_H = 32
D_EH = 32
STEPS = 3
N_GRAPHS = 64

NC, NS = 2, 16
NW = NC * NS
LANES = 128
CHUNK = 1024


def _relu_mm_body(x_ref, w_ref, b_ref, o_ref):
    o_ref[...] = jax.nn.relu(
        jnp.dot(x_ref[...], w_ref[...], preferred_element_type=jnp.float32)
        + b_ref[...])


def _relu_mm(x, w, b, block_rows):
    n, fi = x.shape
    fo = w.shape[1]
    return pl.pallas_call(
        _relu_mm_body,
        grid=(n // block_rows,),
        in_specs=[pl.BlockSpec((block_rows, fi), lambda i: (i, 0)),
                  pl.BlockSpec((fi, fo), lambda i: (0, 0)),
                  pl.BlockSpec((1, fo), lambda i: (0, 0))],
        out_specs=pl.BlockSpec((block_rows, fo), lambda i: (i, 0)),
        out_shape=jax.ShapeDtypeStruct((n, fo), jnp.float32),
    )(x, w, b.reshape(1, fo))


def _relu_mm_t_body(x_ref, w_ref, b_ref, o_ref):
    y = jax.nn.relu(
        jnp.dot(x_ref[...], w_ref[...], preferred_element_type=jnp.float32)
        + b_ref[...])
    o_ref[...] = jnp.concatenate(
        [y.T, jnp.ones((1, y.shape[0]), jnp.float32)], axis=0)


def _relu_mm_t(x, w, b, block_rows):
    n, fi = x.shape
    fo = w.shape[1]
    return pl.pallas_call(
        _relu_mm_t_body,
        grid=(n // block_rows,),
        in_specs=[pl.BlockSpec((block_rows, fi), lambda i: (i, 0)),
                  pl.BlockSpec((fi, fo), lambda i: (0, 0)),
                  pl.BlockSpec((1, fo), lambda i: (0, 0))],
        out_specs=pl.BlockSpec((fo + 1, block_rows), lambda i: (0, i)),
        out_shape=jax.ShapeDtypeStruct((fo + 1, n), jnp.float32),
    )(x, w, b.reshape(1, fo))


def _msg_body(hs_ref, ut_ref, wt_ref, o_ref):
    hst = hs_ref[...].T
    t3 = jnp.dot(wt_ref[...], hst, preferred_element_type=jnp.float32)
    te = hst.shape[1]
    ub = ut_ref[...].reshape(D_EH + 1, 1, te)
    p = t3.reshape(D_EH + 1, D_H, te) * ub
    msg3 = p.sum(axis=0)
    o_ref[...] = msg3.T


def _msg(hs, ut, wt, block_rows):
    e_pad = hs.shape[0]
    return pl.pallas_call(
        _msg_body,
        grid=(e_pad // block_rows,),
        in_specs=[pl.BlockSpec((block_rows, D_H), lambda i: (i, 0)),
                  pl.BlockSpec((D_EH + 1, block_rows), lambda i: (0, i)),
                  pl.BlockSpec(((D_EH + 1) * D_H, D_H), lambda i: (0, 0))],
        out_specs=pl.BlockSpec((block_rows, D_H), lambda i: (i, 0)),
        out_shape=jax.ShapeDtypeStruct((e_pad, D_H), jnp.float32),
    )(hs, ut, wt)


def _gru_body(p_ref0, p_ref1, h_ref, wih_ref, bih_ref, whh_ref, bhh_ref,
              bconv_ref, o_ref):
    agg = p_ref0[0] + p_ref1[0] + bconv_ref[...]
    gi = jnp.dot(agg, wih_ref[...], preferred_element_type=jnp.float32) + bih_ref[...]
    h = h_ref[...]
    gh = jnp.dot(h, whh_ref[...], preferred_element_type=jnp.float32) + bhh_ref[...]
    r = jax.nn.sigmoid(gi[:, :D_H] + gh[:, :D_H])
    z = jax.nn.sigmoid(gi[:, D_H:2 * D_H] + gh[:, D_H:2 * D_H])
    nn = jnp.tanh(gi[:, 2 * D_H:] + r * gh[:, 2 * D_H:])
    o_ref[...] = (1.0 - z) * nn + z * h


def _gru(parts, h, w_ih, b_ih, w_hh, b_hh, b_conv, block_rows):
    n = h.shape[0]
    return pl.pallas_call(
        _gru_body,
        grid=(n // block_rows,),
        in_specs=[pl.BlockSpec((1, block_rows, D_H), lambda i: (0, i, 0)),
                  pl.BlockSpec((1, block_rows, D_H), lambda i: (1, i, 0)),
                  pl.BlockSpec((block_rows, D_H), lambda i: (i, 0)),
                  pl.BlockSpec((D_H, 3 * D_H), lambda i: (0, 0)),
                  pl.BlockSpec((1, 3 * D_H), lambda i: (0, 0)),
                  pl.BlockSpec((D_H, 3 * D_H), lambda i: (0, 0)),
                  pl.BlockSpec((1, 3 * D_H), lambda i: (0, 0)),
                  pl.BlockSpec((1, D_H), lambda i: (0, 0))],
        out_specs=pl.BlockSpec((block_rows, D_H), lambda i: (i, 0)),
        out_shape=jax.ShapeDtypeStruct((n, D_H), jnp.float32),
    )(parts, parts, h, w_ih, b_ih.reshape(1, -1), w_hh, b_hh.reshape(1, -1),
      b_conv.reshape(1, D_H))


def _readout_body(h_ref, n2g_ref, wf1_ref, bf1_ref, wf2_ref, bf2_ref,
                  wf3_ref, bf3_ref, o_ref, acc_ref):
    pi = pl.program_id(0)

    @pl.when(pi == 0)
    def _():
        acc_ref[...] = jnp.zeros_like(acc_ref)

    n2g = n2g_ref[0, 0, :]
    bn = n2g.shape[0]
    gids = lax.broadcasted_iota(jnp.int32, (N_GRAPHS, bn), 0)
    oh = jnp.where(gids == n2g[None, :], 1.0, 0.0)
    acc_ref[...] += jnp.dot(oh, h_ref[...], preferred_element_type=jnp.float32)

    @pl.when(pi == pl.num_programs(0) - 1)
    def _():
        g = acc_ref[...]
        p = jax.nn.relu(jnp.dot(g, wf1_ref[...],
                                preferred_element_type=jnp.float32) + bf1_ref[...])
        p = jax.nn.relu(jnp.dot(p, wf2_ref[...],
                                preferred_element_type=jnp.float32) + bf2_ref[...])
        o_ref[...] = jnp.dot(p, wf3_ref[...],
                             preferred_element_type=jnp.float32) + bf3_ref[...]


def _readout(h, n2g3d, wf1, bf1, wf2, bf2, wf3, bf3, block_rows):
    n = h.shape[0]
    nblk = n // block_rows
    return pl.pallas_call(
        _readout_body,
        grid=(nblk,),
        in_specs=[pl.BlockSpec((block_rows, D_H), lambda i: (i, 0)),
                  pl.BlockSpec((1, 1, block_rows), lambda i: (i, 0, 0)),
                  pl.BlockSpec((D_H, D_H), lambda i: (0, 0)),
                  pl.BlockSpec((1, D_H), lambda i: (0, 0)),
                  pl.BlockSpec((D_H, D_H), lambda i: (0, 0)),
                  pl.BlockSpec((1, D_H), lambda i: (0, 0)),
                  pl.BlockSpec((D_H, 1), lambda i: (0, 0)),
                  pl.BlockSpec((1, 1), lambda i: (0, 0))],
        out_specs=pl.BlockSpec((N_GRAPHS, 1), lambda i: (0, 0)),
        out_shape=jax.ShapeDtypeStruct((N_GRAPHS, 1), jnp.float32),
        scratch_shapes=[pltpu.VMEM((N_GRAPHS, D_H), jnp.float32)],
    )(h, n2g3d, wf1, bf1.reshape(1, D_H), wf2, bf2.reshape(1, D_H),
      wf3, bf3.reshape(1, 1))


def _sc_gather(table, idx2d, n, e_pad):
    rows_per_w = e_pad // NW
    n_chunks = rows_per_w // CHUNK
    stagers = 10
    t_rows = n // stagers
    mesh = plsc.VectorSubcoreMesh(core_axis_name="c", subcore_axis_name="s")

    @functools.partial(
        pl.kernel, mesh=mesh,
        compiler_params=pltpu.CompilerParams(use_tc_tiling_on_sc=False),
        out_type=jax.ShapeDtypeStruct((e_pad, D_H), jnp.float32),
        scratch_types=[pltpu.VMEM((8, LANES), jnp.int32),
                       pltpu.VMEM((CHUNK, D_H), jnp.float32),
                       pltpu.VMEM_SHARED((n, D_H), jnp.float32),
                       pltpu.SemaphoreType.DMA],
    )
    def k(table_hbm, idx_hbm, out_hbm, idx_v, rows_v, table_sh, sem):
        s = lax.axis_index("s")
        wid = lax.axis_index("c") * NS + s

        @pl.when(s < stagers)
        def _():
            toff = pl.multiple_of(s * t_rows, 8)
            pltpu.sync_copy(table_hbm.at[pl.ds(toff, t_rows)],
                            table_sh.at[pl.ds(toff, t_rows)])
        plsc.subcore_barrier()
        for ci in range(n_chunks):
            base = pl.multiple_of(wid * rows_per_w + ci * CHUNK, 8)
            row = pl.multiple_of(wid * (rows_per_w // LANES) + ci * (CHUNK // LANES), 8)
            pltpu.sync_copy(idx_hbm.at[pl.ds(row, 8)], idx_v)
            copies = []
            for j in range(8):
                copies.append(pltpu.async_copy(
                    table_sh.at[idx_v.at[j]],
                    rows_v.at[pl.ds(j * LANES, LANES)], sem))
            for cp in copies:
                cp.wait()
            pltpu.sync_copy(rows_v, out_hbm.at[pl.ds(base, CHUNK)])

    return k(table, idx2d)


def _sc_scatter_add(dst2d, msg, zeros, n, n_pad, e_pad):
    rows_per_w = e_pad // NW
    n_chunks = rows_per_w // CHUNK
    z_rows = n_pad // NS
    writers = 10
    o_rows = n // writers
    mesh = plsc.VectorSubcoreMesh(core_axis_name="c", subcore_axis_name="s")

    @functools.partial(
        pl.kernel, mesh=mesh,
        compiler_params=pltpu.CompilerParams(use_tc_tiling_on_sc=False),
        out_type=jax.ShapeDtypeStruct((NC, n, D_H), jnp.float32),
        scratch_types=[pltpu.VMEM((8, LANES), jnp.int32),
                       pltpu.VMEM((CHUNK, D_H), jnp.float32),
                       pltpu.VMEM_SHARED((n_pad, D_H), jnp.float32),
                       pltpu.SemaphoreType.DMA],
    )
    def k(dst_hbm, msg_hbm, zeros_hbm, out_hbm, idx_v, msg_v, agg_sh, sem):
        c = lax.axis_index("c")
        s = lax.axis_index("s")
        wid = c * NS + s
        zoff = pl.multiple_of(s * z_rows, 8)
        pltpu.sync_copy(zeros_hbm.at[pl.ds(zoff, z_rows)],
                        agg_sh.at[pl.ds(zoff, z_rows)])
        plsc.subcore_barrier()
        for ci in range(n_chunks):
            base = pl.multiple_of(wid * rows_per_w + ci * CHUNK, 8)
            row = pl.multiple_of(wid * (rows_per_w // LANES) + ci * (CHUNK // LANES), 8)
            pltpu.sync_copy(dst_hbm.at[pl.ds(row, 8)], idx_v)
            pltpu.sync_copy(msg_hbm.at[pl.ds(base, CHUNK)], msg_v)
            for j in range(8):
                pltpu.sync_copy(msg_v.at[pl.ds(j * LANES, LANES)],
                                agg_sh.at[idx_v.at[j]], add=True)
        plsc.subcore_barrier()

        @pl.when(s < writers)
        def _():
            ooff = pl.multiple_of(s * o_rows, 8)
            pltpu.sync_copy(agg_sh.at[pl.ds(ooff, o_rows)],
                            out_hbm.at[c].at[pl.ds(ooff, o_rows)])

    return k(dst2d, msg, zeros)


def kernel(x, edge_index, edge_attr, node2graph,
           W_enc, b_enc, We1, be1, We2, be2, b_conv,
           W_ih, b_ih, W_hh, b_hh,
           Wf1, bf1, Wf2, bf2, Wf3, bf3):
    n, _ = x.shape
    e = edge_attr.shape[0]
    e_pad = -(-e // (NW * CHUNK)) * (NW * CHUNK)
    n_pad = -(-(n + 1) // (NS * 8)) * (NS * 8)

    src = jnp.pad(edge_index[0], (0, e_pad - e)).reshape(e_pad // LANES, LANES)
    dst = jnp.pad(edge_index[1], (0, e_pad - e),
                  constant_values=n).reshape(e_pad // LANES, LANES)
    ea_pad = jnp.pad(edge_attr, ((0, e_pad - e), (0, 0)))

    wt = jnp.concatenate(
        [We2.reshape(D_EH, D_H, D_H).transpose(0, 2, 1).reshape(D_EH * D_H, D_H),
         be2.reshape(D_H, D_H).T], axis=0)
    zeros = jnp.zeros((n_pad, D_H), jnp.float32)

    out = _relu_mm(x, W_enc, b_enc, block_rows=2000)
    ut = _relu_mm_t(ea_pad, We1, be1, block_rows=4096)
    h = out
    for _ in range(STEPS):
        hs = _sc_gather(out, src, n, e_pad)
        msg = _msg(hs, ut, wt, block_rows=1024)
        parts = _sc_scatter_add(dst, msg, zeros, n, n_pad, e_pad)
        h = _gru(parts, h, W_ih, b_ih, W_hh, b_hh, b_conv, block_rows=2000)
        out = h

    n2g3d = node2graph.reshape(5, 1, n // 5)
    return _readout(out, n2g3d, Wf1, bf1, Wf2, bf2, Wf3, bf3,
                    block_rows=n // 5)


# speedup vs baseline: 4.1874x; 1.1338x over previous
import functools

import jax
import jax.numpy as jnp
from jax import lax
from jax.experimental import pallas as pl
from jax.experimental.pallas import tpu as pltpu
from jax.experimental.pallas import tpu_sc as plsc

D_H = 32
D_EH = 32
STEPS = 3
N_GRAPHS = 64

NC, NS = 2, 16
NW = NC * NS
LANES = 128
CHUNK = 1024


def _relu_mm_body(x_ref, w_ref, b_ref, o_ref):
    o_ref[...] = jax.nn.relu(
        jnp.dot(x_ref[...], w_ref[...], preferred_element_type=jnp.float32)
        + b_ref[...])


def _relu_mm(x, w, b, block_rows):
    n, fi = x.shape
    fo = w.shape[1]
    return pl.pallas_call(
        _relu_mm_body,
        grid=(n // block_rows,),
        in_specs=[pl.BlockSpec((block_rows, fi), lambda i: (i, 0)),
                  pl.BlockSpec((fi, fo), lambda i: (0, 0)),
                  pl.BlockSpec((1, fo), lambda i: (0, 0))],
        out_specs=pl.BlockSpec((block_rows, fo), lambda i: (i, 0)),
        out_shape=jax.ShapeDtypeStruct((n, fo), jnp.float32),
    )(x, w, b.reshape(1, fo))


def _relu_mm_t_body(x_ref, w_ref, b_ref, o_ref):
    y = jax.nn.relu(
        jnp.dot(x_ref[...], w_ref[...], preferred_element_type=jnp.float32)
        + b_ref[...])
    o_ref[...] = jnp.concatenate(
        [y.T, jnp.ones((1, y.shape[0]), jnp.float32)], axis=0)


def _relu_mm_t(x, w, b, block_rows):
    n, fi = x.shape
    fo = w.shape[1]
    return pl.pallas_call(
        _relu_mm_t_body,
        grid=(n // block_rows,),
        in_specs=[pl.BlockSpec((block_rows, fi), lambda i: (i, 0)),
                  pl.BlockSpec((fi, fo), lambda i: (0, 0)),
                  pl.BlockSpec((1, fo), lambda i: (0, 0))],
        out_specs=pl.BlockSpec((fo + 1, block_rows), lambda i: (0, i)),
        out_shape=jax.ShapeDtypeStruct((fo + 1, n), jnp.float32),
    )(x, w, b.reshape(1, fo))


def _msg_body(hs_ref, ut_ref, wt_ref, o_ref):
    hst = hs_ref[...].T
    t3 = jnp.dot(wt_ref[...], hst, preferred_element_type=jnp.float32)
    te = hst.shape[1]
    ub = ut_ref[...].reshape(D_EH + 1, 1, te)
    p = t3.reshape(D_EH + 1, D_H, te) * ub
    msg3 = p.sum(axis=0)
    o_ref[...] = msg3.T


def _msg(hs, ut, wt, block_rows):
    e_pad = hs.shape[0]
    return pl.pallas_call(
        _msg_body,
        grid=(e_pad // block_rows,),
        in_specs=[pl.BlockSpec((block_rows, D_H), lambda i: (i, 0)),
                  pl.BlockSpec((D_EH + 1, block_rows), lambda i: (0, i)),
                  pl.BlockSpec(((D_EH + 1) * D_H, D_H), lambda i: (0, 0))],
        out_specs=pl.BlockSpec((block_rows, D_H), lambda i: (i, 0)),
        out_shape=jax.ShapeDtypeStruct((e_pad, D_H), jnp.float32),
    )(hs, ut, wt)


def _gru_body(p_ref0, p_ref1, h_ref, wih_ref, bih_ref, whh_ref, bhh_ref,
              bconv_ref, o_ref):
    agg = p_ref0[0] + p_ref1[0] + bconv_ref[...]
    gi = jnp.dot(agg, wih_ref[...], preferred_element_type=jnp.float32) + bih_ref[...]
    h = h_ref[...]
    gh = jnp.dot(h, whh_ref[...], preferred_element_type=jnp.float32) + bhh_ref[...]
    r = jax.nn.sigmoid(gi[:, :D_H] + gh[:, :D_H])
    z = jax.nn.sigmoid(gi[:, D_H:2 * D_H] + gh[:, D_H:2 * D_H])
    nn = jnp.tanh(gi[:, 2 * D_H:] + r * gh[:, 2 * D_H:])
    o_ref[...] = (1.0 - z) * nn + z * h


def _gru(parts, h, w_ih, b_ih, w_hh, b_hh, b_conv, block_rows):
    n = h.shape[0]
    return pl.pallas_call(
        _gru_body,
        grid=(n // block_rows,),
        in_specs=[pl.BlockSpec((1, block_rows, D_H), lambda i: (0, i, 0)),
                  pl.BlockSpec((1, block_rows, D_H), lambda i: (1, i, 0)),
                  pl.BlockSpec((block_rows, D_H), lambda i: (i, 0)),
                  pl.BlockSpec((D_H, 3 * D_H), lambda i: (0, 0)),
                  pl.BlockSpec((1, 3 * D_H), lambda i: (0, 0)),
                  pl.BlockSpec((D_H, 3 * D_H), lambda i: (0, 0)),
                  pl.BlockSpec((1, 3 * D_H), lambda i: (0, 0)),
                  pl.BlockSpec((1, D_H), lambda i: (0, 0))],
        out_specs=pl.BlockSpec((block_rows, D_H), lambda i: (i, 0)),
        out_shape=jax.ShapeDtypeStruct((n, D_H), jnp.float32),
    )(parts, parts, h, w_ih, b_ih.reshape(1, -1), w_hh, b_hh.reshape(1, -1),
      b_conv.reshape(1, D_H))


def _readout_body(h_ref, n2g_ref, wf1_ref, bf1_ref, wf2_ref, bf2_ref,
                  wf3_ref, bf3_ref, o_ref, acc_ref):
    pi = pl.program_id(0)

    @pl.when(pi == 0)
    def _():
        acc_ref[...] = jnp.zeros_like(acc_ref)

    n2g = n2g_ref[0, 0, :]
    bn = n2g.shape[0]
    gids = lax.broadcasted_iota(jnp.int32, (N_GRAPHS, bn), 0)
    oh = jnp.where(gids == n2g[None, :], 1.0, 0.0)
    acc_ref[...] += jnp.dot(oh, h_ref[...], preferred_element_type=jnp.float32)

    @pl.when(pi == pl.num_programs(0) - 1)
    def _():
        g = acc_ref[...]
        p = jax.nn.relu(jnp.dot(g, wf1_ref[...],
                                preferred_element_type=jnp.float32) + bf1_ref[...])
        p = jax.nn.relu(jnp.dot(p, wf2_ref[...],
                                preferred_element_type=jnp.float32) + bf2_ref[...])
        o_ref[...] = jnp.dot(p, wf3_ref[...],
                             preferred_element_type=jnp.float32) + bf3_ref[...]


def _readout(h, n2g3d, wf1, bf1, wf2, bf2, wf3, bf3, block_rows):
    n = h.shape[0]
    nblk = n // block_rows
    return pl.pallas_call(
        _readout_body,
        grid=(nblk,),
        in_specs=[pl.BlockSpec((block_rows, D_H), lambda i: (i, 0)),
                  pl.BlockSpec((1, 1, block_rows), lambda i: (i, 0, 0)),
                  pl.BlockSpec((D_H, D_H), lambda i: (0, 0)),
                  pl.BlockSpec((1, D_H), lambda i: (0, 0)),
                  pl.BlockSpec((D_H, D_H), lambda i: (0, 0)),
                  pl.BlockSpec((1, D_H), lambda i: (0, 0)),
                  pl.BlockSpec((D_H, 1), lambda i: (0, 0)),
                  pl.BlockSpec((1, 1), lambda i: (0, 0))],
        out_specs=pl.BlockSpec((N_GRAPHS, 1), lambda i: (0, 0)),
        out_shape=jax.ShapeDtypeStruct((N_GRAPHS, 1), jnp.float32),
        scratch_shapes=[pltpu.VMEM((N_GRAPHS, D_H), jnp.float32)],
    )(h, n2g3d, wf1, bf1.reshape(1, D_H), wf2, bf2.reshape(1, D_H),
      wf3, bf3.reshape(1, 1))


def _sc_gather(table, idx2d, n, e_pad):
    rows_per_w = e_pad // NW
    n_chunks = rows_per_w // CHUNK
    stagers = 10
    t_rows = n // stagers
    mesh = plsc.VectorSubcoreMesh(core_axis_name="c", subcore_axis_name="s")

    @functools.partial(
        pl.kernel, mesh=mesh,
        compiler_params=pltpu.CompilerParams(use_tc_tiling_on_sc=False),
        out_type=jax.ShapeDtypeStruct((e_pad, D_H), jnp.float32),
        scratch_types=[pltpu.VMEM((8, LANES), jnp.int32),
                       pltpu.VMEM((CHUNK, D_H), jnp.float32),
                       pltpu.VMEM_SHARED((n, D_H), jnp.float32),
                       pltpu.SemaphoreType.DMA],
    )
    def k(table_hbm, idx_hbm, out_hbm, idx_v, rows_v, table_sh, sem):
        s = lax.axis_index("s")
        wid = lax.axis_index("c") * NS + s

        @pl.when(s < stagers)
        def _():
            toff = pl.multiple_of(s * t_rows, 8)
            pltpu.sync_copy(table_hbm.at[pl.ds(toff, t_rows)],
                            table_sh.at[pl.ds(toff, t_rows)])
        plsc.subcore_barrier()
        for ci in range(n_chunks):
            base = pl.multiple_of(wid * rows_per_w + ci * CHUNK, 8)
            row = pl.multiple_of(wid * (rows_per_w // LANES) + ci * (CHUNK // LANES), 8)
            pltpu.sync_copy(idx_hbm.at[pl.ds(row, 8)], idx_v)
            copies = []
            for j in range(8):
                copies.append(pltpu.async_copy(
                    table_sh.at[idx_v.at[j]],
                    rows_v.at[pl.ds(j * LANES, LANES)], sem))
            for cp in copies:
                cp.wait()
            pltpu.sync_copy(rows_v, out_hbm.at[pl.ds(base, CHUNK)])

    return k(table, idx2d)


def _sc_scatter_add(dst2d, msg, zeros, n, n_pad, e_pad):
    rows_per_w = e_pad // NW
    n_chunks = rows_per_w // CHUNK
    z_rows = n_pad // NS
    writers = 10
    o_rows = n // writers
    mesh = plsc.VectorSubcoreMesh(core_axis_name="c", subcore_axis_name="s")

    @functools.partial(
        pl.kernel, mesh=mesh,
        compiler_params=pltpu.CompilerParams(use_tc_tiling_on_sc=False),
        out_type=jax.ShapeDtypeStruct((NC, n, D_H), jnp.float32),
        scratch_types=[pltpu.VMEM((8, LANES), jnp.int32),
                       pltpu.VMEM((CHUNK, D_H), jnp.float32),
                       pltpu.VMEM_SHARED((n_pad, D_H), jnp.float32),
                       pltpu.SemaphoreType.DMA],
    )
    def k(dst_hbm, msg_hbm, zeros_hbm, out_hbm, idx_v, msg_v, agg_sh, sem):
        c = lax.axis_index("c")
        s = lax.axis_index("s")
        wid = c * NS + s
        zoff = pl.multiple_of(s * z_rows, 8)
        pltpu.sync_copy(zeros_hbm.at[pl.ds(zoff, z_rows)],
                        agg_sh.at[pl.ds(zoff, z_rows)])
        plsc.subcore_barrier()
        for ci in range(n_chunks):
            base = pl.multiple_of(wid * rows_per_w + ci * CHUNK, 8)
            row = pl.multiple_of(wid * (rows_per_w // LANES) + ci * (CHUNK // LANES), 8)
            pltpu.sync_copy(dst_hbm.at[pl.ds(row, 8)], idx_v)
            pltpu.sync_copy(msg_hbm.at[pl.ds(base, CHUNK)], msg_v)
            for j in range(8):
                pltpu.sync_copy(msg_v.at[pl.ds(j * LANES, LANES)],
                                agg_sh.at[idx_v.at[j]], add=True)
        plsc.subcore_barrier()

        @pl.when(s < writers)
        def _():
            ooff = pl.multiple_of(s * o_rows, 8)
            pltpu.sync_copy(agg_sh.at[pl.ds(ooff, o_rows)],
                            out_hbm.at[c].at[pl.ds(ooff, o_rows)])

    return k(dst2d, msg, zeros)


def kernel(x, edge_index, edge_attr, node2graph,
           W_enc, b_enc, We1, be1, We2, be2, b_conv,
           W_ih, b_ih, W_hh, b_hh,
           Wf1, bf1, Wf2, bf2, Wf3, bf3):
    n, _ = x.shape
    e = edge_attr.shape[0]
    e_pad = -(-e // (NW * CHUNK)) * (NW * CHUNK)
    n_pad = -(-(n + 1) // (NS * 8)) * (NS * 8)

    src = jnp.pad(edge_index[0], (0, e_pad - e)).reshape(e_pad // LANES, LANES)
    dst = jnp.pad(edge_index[1], (0, e_pad - e),
                  constant_values=n).reshape(e_pad // LANES, LANES)
    ea_pad = jnp.pad(edge_attr, ((0, e_pad - e), (0, 0)))

    wt = jnp.concatenate(
        [We2.reshape(D_EH, D_H, D_H).transpose(0, 2, 1).reshape(D_EH * D_H, D_H),
         be2.reshape(D_H, D_H).T], axis=0)
    zeros = jnp.zeros((n_pad, D_H), jnp.float32)

    out = _relu_mm(x, W_enc, b_enc, block_rows=2000)
    ut = _relu_mm_t(ea_pad, We1, be1, block_rows=4096)
    h = out
    for _ in range(STEPS):
        hs = _sc_gather(out, src, n, e_pad)
        msg = _msg(hs, ut, wt, block_rows=2048)
        parts = _sc_scatter_add(dst, msg, zeros, n, n_pad, e_pad)
        h = _gru(parts, h, W_ih, b_ih, W_hh, b_hh, b_conv, block_rows=2000)
        out = h

    n2g3d = node2graph.reshape(5, 1, n // 5)
    return _readout(out, n2g3d, Wf1, bf1, Wf2, bf2, Wf3, bf3,
                    block_rows=n // 5)
